# baseline (speedup 1.0000x reference)
import functools

import jax
import jax.numpy as jnp
from jax.experimental import pallas as pl
from jax.experimental.pallas import tpu as pltpu

_F32 = jnp.float32


def _prep_body(rot_ref, t_ref, feat_ref, wc_ref, bc_ref, bp_ref,
               a12_ref, bias_ref):
    q = rot_ref[...]
    norm = jnp.sqrt(jnp.sum(q * q, axis=1, keepdims=True))
    qn = q / jnp.maximum(norm, 1e-8)
    qw = qn[:, 0:1]
    qx = qn[:, 1:2]
    qy = qn[:, 2:3]
    qz = qn[:, 3:4]
    xx = qx * qx
    yy = qy * qy
    zz = qz * qz
    xy = qx * qy
    xz = qx * qz
    yz = qy * qz
    wx = qw * qx
    wy = qw * qy
    wz = qw * qz
    one = jnp.ones_like(qw)
    r00 = one - 2.0 * (yy + zz)
    r01 = 2.0 * (xy + wz)
    r02 = 2.0 * (xz - wy)
    r10 = 2.0 * (xy - wz)
    r11 = one - 2.0 * (xx + zz)
    r12 = 2.0 * (yz + wx)
    r20 = 2.0 * (xz + wy)
    r21 = 2.0 * (yz - wx)
    r22 = one - 2.0 * (xx + yy)
    t = t_ref[...]
    tx = t[:, 0:1]
    ty = t[:, 1:2]
    tz = t[:, 2:3]
    c0 = -(r00 * tx + r01 * ty + r02 * tz)
    c1 = -(r10 * tx + r11 * ty + r12 * tz)
    c2 = -(r20 * tx + r21 * ty + r22 * tz)
    a12_ref[...] = jnp.concatenate(
        [r00, r10, r20, r01, r11, r21, r02, r12, r22, c0, c1, c2], axis=1)
    bias_ref[...] = (
        jnp.dot(feat_ref[...], wc_ref[...], preferred_element_type=_F32)
        + bc_ref[...] + bp_ref[...])


def _main_body(paug_ref, a_ref, bias_ref, maskf_ref, wp_ref, w1_ref, b1_ref,
               w2_ref, b2_ref, wout_ref, bout_ref, ptm_ref, imp_ref, *, T, M):
    paug = paug_ref[0]
    a = a_ref[0]
    ptm = jnp.dot(paug, a, preferred_element_type=_F32)
    ptm_ref[0] = ptm
    wp = wp_ref[...]
    bias = bias_ref[0]
    nets = []
    for m in range(M):
        pm = ptm[:, m * 3:(m + 1) * 3]
        nets.append(jnp.dot(pm, wp, preferred_element_type=_F32)
                    + bias[m:m + 1, :])
    net = jnp.concatenate(nets, axis=0)
    h = jnp.dot(jnp.maximum(net, 0.0), w1_ref[...],
                preferred_element_type=_F32) + b1_ref[...]
    h = jnp.dot(jnp.maximum(h, 0.0), w2_ref[...],
                preferred_element_type=_F32) + b2_ref[...]
    net = net + h
    r = jnp.maximum(net, 0.0) * wout_ref[...]
    occ_flat = jnp.sum(r, axis=1, keepdims=True) + bout_ref[...]
    occ = jnp.concatenate(
        [occ_flat[m * T:(m + 1) * T] for m in range(M)], axis=1)
    imp_ref[0] = maskf_ref[0] * jax.nn.sigmoid(10.0 * occ)


def kernel(ray_points, translations, rotations, part_shape_features,
           points_mask, W_p, b_p, W_c, b_c, W1, b1, W2, b2, W_out, b_out):
    B, N, P, _ = ray_points.shape
    M = translations.shape[1]
    C = part_shape_features.shape[-1]
    H = W_p.shape[1]
    Q = N * P

    a12, bias = pl.pallas_call(
        _prep_body,
        out_shape=(
            jax.ShapeDtypeStruct((B * M, 12), _F32),
            jax.ShapeDtypeStruct((B * M, H), _F32),
        ),
    )(
        rotations.reshape(B * M, 4),
        translations.reshape(B * M, 3),
        part_shape_features.reshape(B * M, C),
        W_c,
        b_c.reshape(1, H),
        b_p.reshape(1, H),
    )
    a = a12.reshape(B, M, 4, 3).transpose(0, 2, 1, 3).reshape(B, 4, M * 3)

    pts = ray_points.reshape(B, Q, 3)
    paug = jnp.concatenate([pts, jnp.ones((B, Q, 1), _F32)], axis=-1)
    maskf = points_mask.reshape(B, Q, M).astype(_F32)

    T = 512
    grid = (B, Q // T)
    ptm, imp = pl.pallas_call(
        functools.partial(_main_body, T=T, M=M),
        grid=grid,
        in_specs=[
            pl.BlockSpec((1, T, 4), lambda b, i: (b, i, 0)),
            pl.BlockSpec((1, 4, M * 3), lambda b, i: (b, 0, 0)),
            pl.BlockSpec((1, M, H), lambda b, i: (b, 0, 0)),
            pl.BlockSpec((1, T, M), lambda b, i: (b, i, 0)),
            pl.BlockSpec((3, H), lambda b, i: (0, 0)),
            pl.BlockSpec((H, H), lambda b, i: (0, 0)),
            pl.BlockSpec((1, H), lambda b, i: (0, 0)),
            pl.BlockSpec((H, H), lambda b, i: (0, 0)),
            pl.BlockSpec((1, H), lambda b, i: (0, 0)),
            pl.BlockSpec((1, H), lambda b, i: (0, 0)),
            pl.BlockSpec((1, 1), lambda b, i: (0, 0)),
        ],
        out_specs=[
            pl.BlockSpec((1, T, M * 3), lambda b, i: (b, i, 0)),
            pl.BlockSpec((1, T, M), lambda b, i: (b, i, 0)),
        ],
        out_shape=(
            jax.ShapeDtypeStruct((B, Q, M * 3), _F32),
            jax.ShapeDtypeStruct((B, Q, M), _F32),
        ),
        compiler_params=pltpu.CompilerParams(
            dimension_semantics=("parallel", "parallel")),
    )(
        paug, a, bias.reshape(B, M, H), maskf, W_p, W1, b1.reshape(1, H),
        W2, b2.reshape(1, H), W_out.reshape(1, H), b_out.reshape(1, 1),
    )

    implicit_field = imp.reshape(B, N, P, M)
    points_transformed = ptm.reshape(B, N, P, M * 3)
    return implicit_field, points_transformed


# speedup vs baseline: 1.3171x; 1.3171x over previous
import functools

import jax
import jax.numpy as jnp
from jax.experimental import pallas as pl
from jax.experimental.pallas import tpu as pltpu

_F32 = jnp.float32


def _prep1_body(rot_ref, t_ref, a12_ref):
    q = rot_ref[...]
    norm = jnp.sqrt(jnp.sum(q * q, axis=1, keepdims=True))
    qn = q / jnp.maximum(norm, 1e-8)
    qw = qn[:, 0:1]
    qx = qn[:, 1:2]
    qy = qn[:, 2:3]
    qz = qn[:, 3:4]
    xx = qx * qx
    yy = qy * qy
    zz = qz * qz
    xy = qx * qy
    xz = qx * qz
    yz = qy * qz
    wx = qw * qx
    wy = qw * qy
    wz = qw * qz
    one = jnp.ones_like(qw)
    r00 = one - 2.0 * (yy + zz)
    r01 = 2.0 * (xy + wz)
    r02 = 2.0 * (xz - wy)
    r10 = 2.0 * (xy - wz)
    r11 = one - 2.0 * (xx + zz)
    r12 = 2.0 * (yz + wx)
    r20 = 2.0 * (xz + wy)
    r21 = 2.0 * (yz - wx)
    r22 = one - 2.0 * (xx + yy)
    t = t_ref[...]
    tx = t[:, 0:1]
    ty = t[:, 1:2]
    tz = t[:, 2:3]
    c0 = -(r00 * tx + r01 * ty + r02 * tz)
    c1 = -(r10 * tx + r11 * ty + r12 * tz)
    c2 = -(r20 * tx + r21 * ty + r22 * tz)
    a12_ref[...] = jnp.concatenate(
        [r00, r10, r20, r01, r11, r21, r02, r12, r22, c0, c1, c2], axis=1)


def _prep2_body(abig_ref, wp_ref, feat_ref, wc_ref, bc_ref, bp_ref,
                g_ref, bias_ref):
    g_ref[...] = jnp.dot(abig_ref[...], wp_ref[...],
                         preferred_element_type=_F32)
    bias_ref[...] = (
        jnp.dot(feat_ref[...], wc_ref[...], preferred_element_type=_F32)
        + bc_ref[...] + bp_ref[...])


def _main_body(paug_ref, a_ref, g_ref, bias_ref, maskf_ref, w1_ref, b1_ref,
               w2_ref, b2_ref, wout_ref, bout_ref, ptm_ref, imp_ref, *, M):
    paug = paug_ref[0]
    ptm_ref[0] = jnp.dot(paug, a_ref[0], preferred_element_type=_F32)
    g = g_ref[0]
    bias = bias_ref[0]
    w1 = w1_ref[...]
    b1 = b1_ref[...]
    w2 = w2_ref[...]
    b2 = b2_ref[...]
    wout = wout_ref[...]
    bout = bout_ref[...]
    occ_cols = []
    for m in range(M):
        net = jnp.dot(paug, g[4 * m:4 * m + 4, :],
                      preferred_element_type=_F32) + bias[m:m + 1, :]
        h = jnp.dot(jnp.maximum(net, 0.0), w1,
                    preferred_element_type=_F32) + b1
        h = jnp.dot(jnp.maximum(h, 0.0), w2,
                    preferred_element_type=_F32) + b2
        net = net + h
        occ_cols.append(
            jnp.sum(jnp.maximum(net, 0.0) * wout, axis=1, keepdims=True)
            + bout)
    occ = jnp.concatenate(occ_cols, axis=1)
    imp_ref[0] = maskf_ref[0] * jax.nn.sigmoid(10.0 * occ)


def kernel(ray_points, translations, rotations, part_shape_features,
           points_mask, W_p, b_p, W_c, b_c, W1, b1, W2, b2, W_out, b_out):
    B, N, P, _ = ray_points.shape
    M = translations.shape[1]
    C = part_shape_features.shape[-1]
    H = W_p.shape[1]
    Q = N * P

    a12 = pl.pallas_call(
        _prep1_body,
        out_shape=jax.ShapeDtypeStruct((B * M, 12), _F32),
    )(rotations.reshape(B * M, 4), translations.reshape(B * M, 3))

    abig = a12.reshape(B * M * 4, 3)
    a = a12.reshape(B, M, 4, 3).transpose(0, 2, 1, 3).reshape(B, 4, M * 3)

    g, bias = pl.pallas_call(
        _prep2_body,
        out_shape=(
            jax.ShapeDtypeStruct((B * M * 4, H), _F32),
            jax.ShapeDtypeStruct((B * M, H), _F32),
        ),
    )(abig, W_p, part_shape_features.reshape(B * M, C), W_c,
      b_c.reshape(1, H), b_p.reshape(1, H))

    pts = ray_points.reshape(B, Q, 3)
    paug = jnp.concatenate([pts, jnp.ones((B, Q, 1), _F32)], axis=-1)
    maskf = points_mask.reshape(B, Q, M).astype(_F32)

    T = 512
    grid = (B, Q // T)
    ptm, imp = pl.pallas_call(
        functools.partial(_main_body, M=M),
        grid=grid,
        in_specs=[
            pl.BlockSpec((1, T, 4), lambda b, i: (b, i, 0)),
            pl.BlockSpec((1, 4, M * 3), lambda b, i: (b, 0, 0)),
            pl.BlockSpec((1, M * 4, H), lambda b, i: (b, 0, 0)),
            pl.BlockSpec((1, M, H), lambda b, i: (b, 0, 0)),
            pl.BlockSpec((1, T, M), lambda b, i: (b, i, 0)),
            pl.BlockSpec((H, H), lambda b, i: (0, 0)),
            pl.BlockSpec((1, H), lambda b, i: (0, 0)),
            pl.BlockSpec((H, H), lambda b, i: (0, 0)),
            pl.BlockSpec((1, H), lambda b, i: (0, 0)),
            pl.BlockSpec((1, H), lambda b, i: (0, 0)),
            pl.BlockSpec((1, 1), lambda b, i: (0, 0)),
        ],
        out_specs=[
            pl.BlockSpec((1, T, M * 3), lambda b, i: (b, i, 0)),
            pl.BlockSpec((1, T, M), lambda b, i: (b, i, 0)),
        ],
        out_shape=(
            jax.ShapeDtypeStruct((B, Q, M * 3), _F32),
            jax.ShapeDtypeStruct((B, Q, M), _F32),
        ),
        compiler_params=pltpu.CompilerParams(
            dimension_semantics=("parallel", "parallel")),
    )(
        paug, a, g.reshape(B, M * 4, H), bias.reshape(B, M, H), maskf,
        W1, b1.reshape(1, H), W2, b2.reshape(1, H), W_out.reshape(1, H),
        b_out.reshape(1, 1),
    )

    implicit_field = imp.reshape(B, N, P, M)
    points_transformed = ptm.reshape(B, N, P, M * 3)
    return implicit_field, points_transformed
